# baseline (speedup 1.0000x reference)
import functools

import jax
import jax.numpy as jnp
from jax import lax
from jax.experimental import pallas as pl
from jax.experimental.pallas import tpu as pltpu
from jax.experimental.pallas import tpu_sc as plsc

N = 10000
NROWS = 10240
NC, NS = 2, 16
NW = NC * NS
CH = 128
NCHUNK = 2560
TOTAL_E = NCHUNK * CH
EPT1 = NCHUNK // NS
HEPT1 = EPT1 // 2
EPT2 = NCHUNK // NW
RPT = NROWS // NS
D1 = 128
DH = 64
D2 = 16
NBUF = 4
BLK = 1280
GRID = NROWS // BLK

_MESH = plsc.VectorSubcoreMesh(core_axis_name="c", subcore_axis_name="s")


def _edge_pipeline(y_hbm, src_v, dst_v, rows_v, agg_sh, sgs, sss, nchunks):
    for b in range(NBUF):
        pltpu.async_copy(y_hbm.at[src_v.at[b]], rows_v.at[b], sgs[b])
    ngroups = nchunks // NBUF

    def group(g, carry):
        q0 = NBUF * g
        descs = []
        for b in range(NBUF):
            q = q0 + b
            pltpu.make_async_copy(y_hbm.at[src_v.at[q]], rows_v.at[b],
                                  sgs[b]).wait()
            descs.append(pltpu.async_copy(rows_v.at[b],
                                          agg_sh.at[dst_v.at[q]],
                                          sss[b], add=True))
        for b in range(NBUF):
            descs[b].wait()

            @pl.when(g < ngroups - 1)
            def _(b=b):
                pltpu.async_copy(y_hbm.at[src_v.at[q0 + b + NBUF]],
                                 rows_v.at[b], sgs[b])

        return carry

    lax.fori_loop(0, ngroups, group, 0)


def _readout(agg_sh, out_hbm, rows_v, c, s, sems, d):
    nk = RPT // CH
    for k in range(nk):
        b = k % 2
        if k >= 2:
            pltpu.make_async_copy(
                rows_v.at[b],
                out_hbm.at[pl.ds(c * NROWS + s * RPT + (k - 2) * CH, CH)],
                sems[b]).wait()
        pltpu.sync_copy(agg_sh.at[pl.ds(s * RPT + k * CH, CH)], rows_v.at[b])
        pltpu.async_copy(rows_v.at[b],
                         out_hbm.at[pl.ds(c * NROWS + s * RPT + k * CH, CH)],
                         sems[b])
    for k in range(nk - 2, nk):
        b = k % 2
        pltpu.make_async_copy(
            rows_v.at[b],
            out_hbm.at[pl.ds(c * NROWS + s * RPT + k * CH, CH)],
            sems[b]).wait()


@functools.partial(
    pl.kernel,
    compiler_params=pltpu.CompilerParams(use_tc_tiling_on_sc=False),
    out_type=jax.ShapeDtypeStruct((NC * NROWS, DH), jnp.float32),
    mesh=_MESH,
    scratch_types=[
        pltpu.VMEM((HEPT1, CH), jnp.int32),
        pltpu.VMEM((HEPT1, CH), jnp.int32),
        pltpu.VMEM((NBUF, CH, DH), jnp.float32),
        pltpu.VMEM_SHARED((NROWS, DH), jnp.float32),
    ] + [pltpu.SemaphoreType.DMA] * (2 * NBUF),
)
def _sc_agg_l1(ya_hbm, yb_hbm, srcs_hbm, dsts_hbm, zeros_hbm, out_hbm,
               src_v, dst_v, rows_v, agg_sh, *sems):
    sgs, sss = sems[:NBUF], sems[NBUF:]
    c = lax.axis_index("c")
    s = lax.axis_index("s")
    rsl = pl.ds(s * RPT, RPT)
    pltpu.sync_copy(zeros_hbm.at[rsl], agg_sh.at[rsl])
    plsc.subcore_barrier()

    def run(y_hbm):
        for ph in range(2):
            base = s * EPT1 + ph * HEPT1
            pltpu.sync_copy(srcs_hbm.at[pl.ds(base, HEPT1)], src_v)
            pltpu.sync_copy(dsts_hbm.at[pl.ds(base, HEPT1)], dst_v)
            _edge_pipeline(y_hbm, src_v, dst_v, rows_v, agg_sh,
                           sgs, sss, HEPT1)

    @pl.when(c == 0)
    def _():
        run(ya_hbm)

    @pl.when(c == 1)
    def _():
        run(yb_hbm)

    plsc.subcore_barrier()
    _readout(agg_sh, out_hbm, rows_v, c, s, sgs, DH)


@functools.partial(
    pl.kernel,
    compiler_params=pltpu.CompilerParams(use_tc_tiling_on_sc=False),
    out_type=jax.ShapeDtypeStruct((NC * NROWS, D2), jnp.float32),
    mesh=_MESH,
    scratch_types=[
        pltpu.VMEM((EPT2, CH), jnp.int32),
        pltpu.VMEM((EPT2, CH), jnp.int32),
        pltpu.VMEM((NBUF, CH, D2), jnp.float32),
        pltpu.VMEM_SHARED((NROWS, D2), jnp.float32),
    ] + [pltpu.SemaphoreType.DMA] * (2 * NBUF),
)
def _sc_agg_l2(y_hbm, srcs_hbm, dsts_hbm, zeros_hbm, out_hbm,
               src_v, dst_v, rows_v, agg_sh, *sems):
    sgs, sss = sems[:NBUF], sems[NBUF:]
    c = lax.axis_index("c")
    s = lax.axis_index("s")
    wid = c * NS + s
    pltpu.sync_copy(srcs_hbm.at[pl.ds(wid * EPT2, EPT2)], src_v)
    pltpu.sync_copy(dsts_hbm.at[pl.ds(wid * EPT2, EPT2)], dst_v)
    rsl = pl.ds(s * RPT, RPT)
    pltpu.sync_copy(zeros_hbm.at[rsl], agg_sh.at[rsl])
    plsc.subcore_barrier()
    _edge_pipeline(y_hbm, src_v, dst_v, rows_v, agg_sh, sgs, sss, EPT2)
    plsc.subcore_barrier()
    _readout(agg_sh, out_hbm, rows_v, c, s, sgs, D2)


@functools.partial(
    pl.kernel,
    compiler_params=pltpu.CompilerParams(use_tc_tiling_on_sc=False),
    out_type=jax.ShapeDtypeStruct((NC * NROWS, D2), jnp.float32),
    mesh=_MESH,
    scratch_types=[
        pltpu.VMEM((EPT2, CH), jnp.int32),
        pltpu.VMEM((CH, D2), jnp.float32),
        pltpu.VMEM((2, CH, D2), jnp.float32),
        pltpu.VMEM_SHARED((NROWS, D2), jnp.float32),
    ] + [pltpu.SemaphoreType.DMA] * NBUF,
)
def _sc_deg(dsts_hbm, ones_hbm, zeros_hbm, out_hbm,
            dst_v, ones_v, bounce_v, agg_sh, *sems):
    c = lax.axis_index("c")
    s = lax.axis_index("s")
    wid = c * NS + s
    pltpu.sync_copy(dsts_hbm.at[pl.ds(wid * EPT2, EPT2)], dst_v)
    pltpu.sync_copy(ones_hbm, ones_v)
    rsl = pl.ds(s * RPT, RPT)
    pltpu.sync_copy(zeros_hbm.at[rsl], agg_sh.at[rsl])
    plsc.subcore_barrier()

    for b in range(NBUF):
        pltpu.async_copy(ones_v, agg_sh.at[dst_v.at[b]], sems[b], add=True)
    ngroups = EPT2 // NBUF

    def group(g, carry):
        q0 = NBUF * g
        for b in range(NBUF):
            pltpu.make_async_copy(ones_v, agg_sh.at[dst_v.at[q0 + b]],
                                  sems[b]).wait()

            @pl.when(g < ngroups - 1)
            def _(b=b):
                pltpu.async_copy(ones_v, agg_sh.at[dst_v.at[q0 + b + NBUF]],
                                 sems[b], add=True)

        return carry

    lax.fori_loop(0, ngroups, group, 0)
    plsc.subcore_barrier()
    _readout(agg_sh, out_hbm, bounce_v, c, s, sems[:2], D2)


def _tc1_body(deg0_ref, deg1_ref, x_ref, w_ref, ya_ref, yb_ref, dinv_ref):
    d = deg0_ref[:, 0:1] + deg1_ref[:, 0:1]
    dinv = jnp.where(d > 0, lax.rsqrt(d), 0.0)
    xw = jnp.dot(x_ref[...], w_ref[...], preferred_element_type=jnp.float32)
    y = dinv * xw
    ya_ref[...] = y[:, :DH]
    yb_ref[...] = y[:, DH:]
    dinv_ref[...] = jnp.broadcast_to(dinv, (BLK, D2))


def _tc2_body(aa_ref, ab_ref, ya_ref, yb_ref, dinv_ref, w2_ref, b1_ref, y2_ref):
    a = jnp.concatenate([aa_ref[...] + ya_ref[...],
                         ab_ref[...] + yb_ref[...]], axis=1)
    dv = dinv_ref[:, 0:1]
    h = jnp.maximum(dv * a + b1_ref[...], 0.0)
    y2_ref[...] = dv * jnp.dot(h, w2_ref[...], preferred_element_type=jnp.float32)


def _tc3_body(a0_ref, a1_ref, y2_ref, dinv_ref, b2_ref, out_ref):
    z = dinv_ref[:, 0:1] * (a0_ref[...] + a1_ref[...] + y2_ref[...]) + b2_ref[...]
    z0 = z[:, 0:1]
    z1 = z[:, 1:2]
    m = jnp.maximum(z0, z1)
    e0 = jnp.exp(z0 - m)
    e1 = jnp.exp(z1 - m)
    den = e0 + e1
    out_ref[...] = jnp.concatenate(
        [e0 / den, e1 / den, jnp.zeros((BLK, D2 - 2), jnp.float32)], axis=1)


def _row_spec(d):
    return pl.BlockSpec((BLK, d), lambda i: (i, 0))


def _full_spec(shape):
    nd = len(shape)
    return pl.BlockSpec(shape, lambda i: (0,) * nd)


def kernel(x, edge_index, W1, b1, W2, b2):
    e32 = edge_index.astype(jnp.int32)
    src = e32[0]
    dst = e32[1]
    pad = TOTAL_E - src.shape[0]
    srcp = jnp.concatenate([src, jnp.zeros((pad,), jnp.int32)]).reshape(NCHUNK, CH)
    dstp = jnp.concatenate([dst, jnp.full((pad,), N, jnp.int32)]).reshape(NCHUNK, CH)
    xp = jnp.pad(x, ((0, NROWS - N), (0, 0)))
    w2p = jnp.pad(W2, ((0, 0), (0, D2 - W2.shape[1])))
    b1r = b1.reshape(1, D1)
    b2r = jnp.pad(b2, (0, D2 - 2)).reshape(1, D2)
    zeros_dh = jnp.zeros((NROWS, DH), jnp.float32)
    zeros_d2 = jnp.zeros((NROWS, D2), jnp.float32)
    ones_d2 = jnp.ones((CH, D2), jnp.float32)

    degp = _sc_deg(dstp, ones_d2, zeros_d2)
    deg0, deg1 = degp[:NROWS], degp[NROWS:]

    y1a, y1b, dinv = pl.pallas_call(
        _tc1_body,
        grid=(GRID,),
        in_specs=[_row_spec(D2), _row_spec(D2), _row_spec(D1),
                  _full_spec((D1, D1))],
        out_specs=[_row_spec(DH), _row_spec(DH), _row_spec(D2)],
        out_shape=[jax.ShapeDtypeStruct((NROWS, DH), jnp.float32),
                   jax.ShapeDtypeStruct((NROWS, DH), jnp.float32),
                   jax.ShapeDtypeStruct((NROWS, D2), jnp.float32)],
    )(deg0, deg1, xp, W1)

    agg1 = _sc_agg_l1(y1a, y1b, srcp, dstp, zeros_dh)
    agg1a, agg1b = agg1[:NROWS], agg1[NROWS:]

    y2 = pl.pallas_call(
        _tc2_body,
        grid=(GRID,),
        in_specs=[_row_spec(DH), _row_spec(DH), _row_spec(DH), _row_spec(DH),
                  _row_spec(D2), _full_spec((D1, D2)), _full_spec((1, D1))],
        out_specs=_row_spec(D2),
        out_shape=jax.ShapeDtypeStruct((NROWS, D2), jnp.float32),
    )(agg1a, agg1b, y1a, y1b, dinv, w2p, b1r)

    agg2 = _sc_agg_l2(y2, srcp, dstp, zeros_d2)
    a2_0, a2_1 = agg2[:NROWS], agg2[NROWS:]

    out = pl.pallas_call(
        _tc3_body,
        grid=(GRID,),
        in_specs=[_row_spec(D2), _row_spec(D2), _row_spec(D2), _row_spec(D2),
                  _full_spec((1, D2))],
        out_specs=_row_spec(D2),
        out_shape=jax.ShapeDtypeStruct((NROWS, D2), jnp.float32),
    )(a2_0, a2_1, y2, dinv, b2r)

    return out[:N, :2]


# speedup vs baseline: 1.5280x; 1.5280x over previous
import functools

import jax
import jax.numpy as jnp
from jax import lax
from jax.experimental import pallas as pl
from jax.experimental.pallas import tpu as pltpu
from jax.experimental.pallas import tpu_sc as plsc

N = 10000
NROWS = 10240
NC, NS = 2, 16
NW = NC * NS
CH = 128
NCHUNK = 2560
TOTAL_E = NCHUNK * CH
EPT1 = NCHUNK // NS
QEPT1 = EPT1 // 4
EPT2 = NCHUNK // NW
RPT = NROWS // NS
D1 = 128
DH = 64
D2 = 16
NBUF = 4
BLK = 1280
GRID = NROWS // BLK

_MESH = plsc.VectorSubcoreMesh(core_axis_name="c", subcore_axis_name="s")


def _edge_pipeline(y_src, src_v, dst_v, rows_v, agg_sh, sgs, sss, nchunks,
                   nbuf):
    for b in range(nbuf):
        pltpu.async_copy(y_src.at[src_v.at[b]], rows_v.at[b], sgs[b])
    ngroups = nchunks // nbuf

    def group(g, carry):
        q0 = nbuf * g
        descs = []
        for b in range(nbuf):
            q = q0 + b
            pltpu.make_async_copy(y_src.at[src_v.at[q]], rows_v.at[b],
                                  sgs[b]).wait()
            descs.append(pltpu.async_copy(rows_v.at[b],
                                          agg_sh.at[dst_v.at[q]],
                                          sss[b], add=True))
        for b in range(nbuf):
            descs[b].wait()

            @pl.when(g < ngroups - 1)
            def _(b=b):
                pltpu.async_copy(y_src.at[src_v.at[q0 + b + nbuf]],
                                 rows_v.at[b], sgs[b])

        return carry

    lax.fori_loop(0, ngroups, group, 0)


def _readout(agg_sh, out_hbm, rows_v, c, s, sems, d):
    nk = RPT // CH
    for k in range(nk):
        b = k % 2
        if k >= 2:
            pltpu.make_async_copy(
                rows_v.at[b],
                out_hbm.at[pl.ds(c * NROWS + s * RPT + (k - 2) * CH, CH)],
                sems[b]).wait()
        pltpu.sync_copy(agg_sh.at[pl.ds(s * RPT + k * CH, CH)], rows_v.at[b])
        pltpu.async_copy(rows_v.at[b],
                         out_hbm.at[pl.ds(c * NROWS + s * RPT + k * CH, CH)],
                         sems[b])
    for k in range(nk - 2, nk):
        b = k % 2
        pltpu.make_async_copy(
            rows_v.at[b],
            out_hbm.at[pl.ds(c * NROWS + s * RPT + k * CH, CH)],
            sems[b]).wait()


@functools.partial(
    pl.kernel,
    compiler_params=pltpu.CompilerParams(use_tc_tiling_on_sc=False),
    out_type=jax.ShapeDtypeStruct((NC * NROWS, DH), jnp.float32),
    mesh=_MESH,
    scratch_types=[
        pltpu.VMEM((QEPT1, CH), jnp.int32),
        pltpu.VMEM((QEPT1, CH), jnp.int32),
        pltpu.VMEM((2, CH, DH), jnp.float32),
        pltpu.VMEM_SHARED((NROWS, DH), jnp.float32),
        pltpu.VMEM_SHARED((NROWS, DH), jnp.float32),
    ] + [pltpu.SemaphoreType.DMA] * 4,
)
def _sc_agg_l1(ya_hbm, yb_hbm, srcs_hbm, dsts_hbm, zeros_hbm, out_hbm,
               src_v, dst_v, rows_v, y_sh, agg_sh, *sems):
    sgs, sss = sems[:2], sems[2:]
    c = lax.axis_index("c")
    s = lax.axis_index("s")
    rsl = pl.ds(s * RPT, RPT)
    @pl.when(c == 0)
    def _():
        pltpu.sync_copy(ya_hbm.at[rsl], y_sh.at[rsl])

    @pl.when(c == 1)
    def _():
        pltpu.sync_copy(yb_hbm.at[rsl], y_sh.at[rsl])

    pltpu.sync_copy(zeros_hbm.at[rsl], agg_sh.at[rsl])
    plsc.subcore_barrier()

    for ph in range(4):
        base = s * EPT1 + ph * QEPT1
        pltpu.sync_copy(srcs_hbm.at[pl.ds(base, QEPT1)], src_v)
        pltpu.sync_copy(dsts_hbm.at[pl.ds(base, QEPT1)], dst_v)
        _edge_pipeline(y_sh, src_v, dst_v, rows_v, agg_sh,
                       sgs, sss, QEPT1, 2)

    plsc.subcore_barrier()
    _readout(agg_sh, out_hbm, rows_v, c, s, sgs, DH)


@functools.partial(
    pl.kernel,
    compiler_params=pltpu.CompilerParams(use_tc_tiling_on_sc=False),
    out_type=jax.ShapeDtypeStruct((NC * NROWS, D2), jnp.float32),
    mesh=_MESH,
    scratch_types=[
        pltpu.VMEM((EPT2, CH), jnp.int32),
        pltpu.VMEM((EPT2, CH), jnp.int32),
        pltpu.VMEM((NBUF, CH, D2), jnp.float32),
        pltpu.VMEM_SHARED((NROWS, D2), jnp.float32),
        pltpu.VMEM_SHARED((NROWS, D2), jnp.float32),
    ] + [pltpu.SemaphoreType.DMA] * (2 * NBUF),
)
def _sc_agg_l2(y_hbm, srcs_hbm, dsts_hbm, zeros_hbm, out_hbm,
               src_v, dst_v, rows_v, y_sh, agg_sh, *sems):
    sgs, sss = sems[:NBUF], sems[NBUF:]
    c = lax.axis_index("c")
    s = lax.axis_index("s")
    wid = c * NS + s
    pltpu.sync_copy(srcs_hbm.at[pl.ds(wid * EPT2, EPT2)], src_v)
    pltpu.sync_copy(dsts_hbm.at[pl.ds(wid * EPT2, EPT2)], dst_v)
    rsl = pl.ds(s * RPT, RPT)
    pltpu.sync_copy(y_hbm.at[rsl], y_sh.at[rsl])
    pltpu.sync_copy(zeros_hbm.at[rsl], agg_sh.at[rsl])
    plsc.subcore_barrier()
    _edge_pipeline(y_sh, src_v, dst_v, rows_v, agg_sh, sgs, sss, EPT2, NBUF)
    plsc.subcore_barrier()
    _readout(agg_sh, out_hbm, rows_v, c, s, sgs, D2)


@functools.partial(
    pl.kernel,
    compiler_params=pltpu.CompilerParams(use_tc_tiling_on_sc=False),
    out_type=jax.ShapeDtypeStruct((NC * NROWS, D2), jnp.float32),
    mesh=_MESH,
    scratch_types=[
        pltpu.VMEM((EPT2, CH), jnp.int32),
        pltpu.VMEM((CH, D2), jnp.float32),
        pltpu.VMEM((2, CH, D2), jnp.float32),
        pltpu.VMEM_SHARED((NROWS, D2), jnp.float32),
    ] + [pltpu.SemaphoreType.DMA] * NBUF,
)
def _sc_deg(dsts_hbm, ones_hbm, zeros_hbm, out_hbm,
            dst_v, ones_v, bounce_v, agg_sh, *sems):
    c = lax.axis_index("c")
    s = lax.axis_index("s")
    wid = c * NS + s
    pltpu.sync_copy(dsts_hbm.at[pl.ds(wid * EPT2, EPT2)], dst_v)
    pltpu.sync_copy(ones_hbm, ones_v)
    rsl = pl.ds(s * RPT, RPT)
    pltpu.sync_copy(zeros_hbm.at[rsl], agg_sh.at[rsl])
    plsc.subcore_barrier()

    for b in range(NBUF):
        pltpu.async_copy(ones_v, agg_sh.at[dst_v.at[b]], sems[b], add=True)
    ngroups = EPT2 // NBUF

    def group(g, carry):
        q0 = NBUF * g
        for b in range(NBUF):
            pltpu.make_async_copy(ones_v, agg_sh.at[dst_v.at[q0 + b]],
                                  sems[b]).wait()

            @pl.when(g < ngroups - 1)
            def _(b=b):
                pltpu.async_copy(ones_v, agg_sh.at[dst_v.at[q0 + b + NBUF]],
                                 sems[b], add=True)

        return carry

    lax.fori_loop(0, ngroups, group, 0)
    plsc.subcore_barrier()
    _readout(agg_sh, out_hbm, bounce_v, c, s, sems[:2], D2)


def _tc1_body(deg0_ref, deg1_ref, x_ref, w_ref, ya_ref, yb_ref, dinv_ref):
    d = deg0_ref[:, 0:1] + deg1_ref[:, 0:1]
    dinv = jnp.where(d > 0, lax.rsqrt(d), 0.0)
    xw = jnp.dot(x_ref[...], w_ref[...], preferred_element_type=jnp.float32)
    y = dinv * xw
    ya_ref[...] = y[:, :DH]
    yb_ref[...] = y[:, DH:]
    dinv_ref[...] = jnp.broadcast_to(dinv, (BLK, D2))


def _tc2_body(aa_ref, ab_ref, ya_ref, yb_ref, dinv_ref, w2_ref, b1_ref, y2_ref):
    a = jnp.concatenate([aa_ref[...] + ya_ref[...],
                         ab_ref[...] + yb_ref[...]], axis=1)
    dv = dinv_ref[:, 0:1]
    h = jnp.maximum(dv * a + b1_ref[...], 0.0)
    y2_ref[...] = dv * jnp.dot(h, w2_ref[...], preferred_element_type=jnp.float32)


def _tc3_body(a0_ref, a1_ref, y2_ref, dinv_ref, b2_ref, out_ref):
    z = dinv_ref[:, 0:1] * (a0_ref[...] + a1_ref[...] + y2_ref[...]) + b2_ref[...]
    z0 = z[:, 0:1]
    z1 = z[:, 1:2]
    m = jnp.maximum(z0, z1)
    e0 = jnp.exp(z0 - m)
    e1 = jnp.exp(z1 - m)
    den = e0 + e1
    out_ref[...] = jnp.concatenate(
        [e0 / den, e1 / den, jnp.zeros((BLK, D2 - 2), jnp.float32)], axis=1)


def _row_spec(d):
    return pl.BlockSpec((BLK, d), lambda i: (i, 0))


def _full_spec(shape):
    nd = len(shape)
    return pl.BlockSpec(shape, lambda i: (0,) * nd)


def kernel(x, edge_index, W1, b1, W2, b2):
    e32 = edge_index.astype(jnp.int32)
    src = e32[0]
    dst = e32[1]
    pad = TOTAL_E - src.shape[0]
    srcp = jnp.concatenate([src, jnp.zeros((pad,), jnp.int32)]).reshape(NCHUNK, CH)
    dstp = jnp.concatenate([dst, jnp.full((pad,), N, jnp.int32)]).reshape(NCHUNK, CH)
    xp = jnp.pad(x, ((0, NROWS - N), (0, 0)))
    w2p = jnp.pad(W2, ((0, 0), (0, D2 - W2.shape[1])))
    b1r = b1.reshape(1, D1)
    b2r = jnp.pad(b2, (0, D2 - 2)).reshape(1, D2)
    zeros_dh = jnp.zeros((NROWS, DH), jnp.float32)
    zeros_d2 = jnp.zeros((NROWS, D2), jnp.float32)
    ones_d2 = jnp.ones((CH, D2), jnp.float32)

    degp = _sc_deg(dstp, ones_d2, zeros_d2)
    deg0, deg1 = degp[:NROWS], degp[NROWS:]

    y1a, y1b, dinv = pl.pallas_call(
        _tc1_body,
        grid=(GRID,),
        in_specs=[_row_spec(D2), _row_spec(D2), _row_spec(D1),
                  _full_spec((D1, D1))],
        out_specs=[_row_spec(DH), _row_spec(DH), _row_spec(D2)],
        out_shape=[jax.ShapeDtypeStruct((NROWS, DH), jnp.float32),
                   jax.ShapeDtypeStruct((NROWS, DH), jnp.float32),
                   jax.ShapeDtypeStruct((NROWS, D2), jnp.float32)],
    )(deg0, deg1, xp, W1)

    agg1 = _sc_agg_l1(y1a, y1b, srcp, dstp, zeros_dh)
    agg1a, agg1b = agg1[:NROWS], agg1[NROWS:]

    y2 = pl.pallas_call(
        _tc2_body,
        grid=(GRID,),
        in_specs=[_row_spec(DH), _row_spec(DH), _row_spec(DH), _row_spec(DH),
                  _row_spec(D2), _full_spec((D1, D2)), _full_spec((1, D1))],
        out_specs=_row_spec(D2),
        out_shape=jax.ShapeDtypeStruct((NROWS, D2), jnp.float32),
    )(agg1a, agg1b, y1a, y1b, dinv, w2p, b1r)

    agg2 = _sc_agg_l2(y2, srcp, dstp, zeros_d2)
    a2_0, a2_1 = agg2[:NROWS], agg2[NROWS:]

    out = pl.pallas_call(
        _tc3_body,
        grid=(GRID,),
        in_specs=[_row_spec(D2), _row_spec(D2), _row_spec(D2), _row_spec(D2),
                  _full_spec((1, D2))],
        out_specs=_row_spec(D2),
        out_shape=jax.ShapeDtypeStruct((NROWS, D2), jnp.float32),
    )(a2_0, a2_1, y2, dinv, b2r)

    return out[:N, :2]


# speedup vs baseline: 1.5326x; 1.0030x over previous
import functools

import jax
import jax.numpy as jnp
from jax import lax
from jax.experimental import pallas as pl
from jax.experimental.pallas import tpu as pltpu
from jax.experimental.pallas import tpu_sc as plsc

N = 10000
NROWS = 10240
NC, NS = 2, 16
NW = NC * NS
CH = 128
NCHUNK = 2560
TOTAL_E = NCHUNK * CH
EPT1 = NCHUNK // NS
QEPT1 = EPT1 // 4
EPT2 = NCHUNK // NW
RPT = NROWS // NS
D1 = 128
DH = 64
D2 = 16
NBUF = 4
BLK = 1280
GRID = NROWS // BLK

_MESH = plsc.VectorSubcoreMesh(core_axis_name="c", subcore_axis_name="s")


def _edge_pipeline(y_src, src_v, dst_v, rows_v, agg_sh, sgs, sss, nchunks,
                   nbuf):
    for b in range(nbuf):
        pltpu.async_copy(y_src.at[src_v.at[b]], rows_v.at[b], sgs[b])
    ngroups = nchunks // nbuf

    def group(g, carry):
        q0 = nbuf * g
        descs = []
        for b in range(nbuf):
            q = q0 + b
            pltpu.make_async_copy(y_src.at[src_v.at[q]], rows_v.at[b],
                                  sgs[b]).wait()
            descs.append(pltpu.async_copy(rows_v.at[b],
                                          agg_sh.at[dst_v.at[q]],
                                          sss[b], add=True))
        for b in range(nbuf):
            descs[b].wait()

            @pl.when(g < ngroups - 1)
            def _(b=b):
                pltpu.async_copy(y_src.at[src_v.at[q0 + b + nbuf]],
                                 rows_v.at[b], sgs[b])

        return carry

    lax.fori_loop(0, ngroups, group, 0)


def _readout(agg_sh, out_hbm, rows_v, c, s, sems, d):
    nk = RPT // CH
    for k in range(nk):
        b = k % 2
        if k >= 2:
            pltpu.make_async_copy(
                rows_v.at[b],
                out_hbm.at[pl.ds(c * NROWS + s * RPT + (k - 2) * CH, CH)],
                sems[b]).wait()
        pltpu.sync_copy(agg_sh.at[pl.ds(s * RPT + k * CH, CH)], rows_v.at[b])
        pltpu.async_copy(rows_v.at[b],
                         out_hbm.at[pl.ds(c * NROWS + s * RPT + k * CH, CH)],
                         sems[b])
    for k in range(nk - 2, nk):
        b = k % 2
        pltpu.make_async_copy(
            rows_v.at[b],
            out_hbm.at[pl.ds(c * NROWS + s * RPT + k * CH, CH)],
            sems[b]).wait()


@functools.partial(
    pl.kernel,
    compiler_params=pltpu.CompilerParams(use_tc_tiling_on_sc=False, skip_device_barrier=True),
    out_type=jax.ShapeDtypeStruct((NC * NROWS, DH), jnp.float32),
    mesh=_MESH,
    scratch_types=[
        pltpu.VMEM((QEPT1, CH), jnp.int32),
        pltpu.VMEM((QEPT1, CH), jnp.int32),
        pltpu.VMEM((2, CH, DH), jnp.float32),
        pltpu.VMEM_SHARED((NROWS, DH), jnp.float32),
        pltpu.VMEM_SHARED((NROWS, DH), jnp.float32),
    ] + [pltpu.SemaphoreType.DMA] * 4,
)
def _sc_agg_l1(ya_hbm, yb_hbm, srcs_hbm, dsts_hbm, zeros_hbm, out_hbm,
               src_v, dst_v, rows_v, y_sh, agg_sh, *sems):
    sgs, sss = sems[:2], sems[2:]
    c = lax.axis_index("c")
    s = lax.axis_index("s")
    rsl = pl.ds(s * RPT, RPT)
    @pl.when(c == 0)
    def _():
        pltpu.sync_copy(ya_hbm.at[rsl], y_sh.at[rsl])

    @pl.when(c == 1)
    def _():
        pltpu.sync_copy(yb_hbm.at[rsl], y_sh.at[rsl])

    pltpu.sync_copy(zeros_hbm.at[rsl], agg_sh.at[rsl])
    plsc.subcore_barrier()

    for ph in range(4):
        base = s * EPT1 + ph * QEPT1
        pltpu.sync_copy(srcs_hbm.at[pl.ds(base, QEPT1)], src_v)
        pltpu.sync_copy(dsts_hbm.at[pl.ds(base, QEPT1)], dst_v)
        _edge_pipeline(y_sh, src_v, dst_v, rows_v, agg_sh,
                       sgs, sss, QEPT1, 2)

    plsc.subcore_barrier()
    _readout(agg_sh, out_hbm, rows_v, c, s, sgs, DH)


@functools.partial(
    pl.kernel,
    compiler_params=pltpu.CompilerParams(use_tc_tiling_on_sc=False, skip_device_barrier=True),
    out_type=jax.ShapeDtypeStruct((NC * NROWS, D2), jnp.float32),
    mesh=_MESH,
    scratch_types=[
        pltpu.VMEM((EPT2, CH), jnp.int32),
        pltpu.VMEM((EPT2, CH), jnp.int32),
        pltpu.VMEM((NBUF, CH, D2), jnp.float32),
        pltpu.VMEM_SHARED((NROWS, D2), jnp.float32),
        pltpu.VMEM_SHARED((NROWS, D2), jnp.float32),
    ] + [pltpu.SemaphoreType.DMA] * (2 * NBUF),
)
def _sc_agg_l2(y_hbm, srcs_hbm, dsts_hbm, zeros_hbm, out_hbm,
               src_v, dst_v, rows_v, y_sh, agg_sh, *sems):
    sgs, sss = sems[:NBUF], sems[NBUF:]
    c = lax.axis_index("c")
    s = lax.axis_index("s")
    wid = c * NS + s
    pltpu.sync_copy(srcs_hbm.at[pl.ds(wid * EPT2, EPT2)], src_v)
    pltpu.sync_copy(dsts_hbm.at[pl.ds(wid * EPT2, EPT2)], dst_v)
    rsl = pl.ds(s * RPT, RPT)
    pltpu.sync_copy(y_hbm.at[rsl], y_sh.at[rsl])
    pltpu.sync_copy(zeros_hbm.at[rsl], agg_sh.at[rsl])
    plsc.subcore_barrier()
    _edge_pipeline(y_sh, src_v, dst_v, rows_v, agg_sh, sgs, sss, EPT2, NBUF)
    plsc.subcore_barrier()
    _readout(agg_sh, out_hbm, rows_v, c, s, sgs, D2)


@functools.partial(
    pl.kernel,
    compiler_params=pltpu.CompilerParams(use_tc_tiling_on_sc=False, skip_device_barrier=True),
    out_type=jax.ShapeDtypeStruct((NC * NROWS, D2), jnp.float32),
    mesh=_MESH,
    scratch_types=[
        pltpu.VMEM((EPT2, CH), jnp.int32),
        pltpu.VMEM((CH, D2), jnp.float32),
        pltpu.VMEM((2, CH, D2), jnp.float32),
        pltpu.VMEM_SHARED((NROWS, D2), jnp.float32),
    ] + [pltpu.SemaphoreType.DMA] * NBUF,
)
def _sc_deg(dsts_hbm, ones_hbm, zeros_hbm, out_hbm,
            dst_v, ones_v, bounce_v, agg_sh, *sems):
    c = lax.axis_index("c")
    s = lax.axis_index("s")
    wid = c * NS + s
    pltpu.sync_copy(dsts_hbm.at[pl.ds(wid * EPT2, EPT2)], dst_v)
    pltpu.sync_copy(ones_hbm, ones_v)
    rsl = pl.ds(s * RPT, RPT)
    pltpu.sync_copy(zeros_hbm.at[rsl], agg_sh.at[rsl])
    plsc.subcore_barrier()

    for b in range(NBUF):
        pltpu.async_copy(ones_v, agg_sh.at[dst_v.at[b]], sems[b], add=True)
    ngroups = EPT2 // NBUF

    def group(g, carry):
        q0 = NBUF * g
        for b in range(NBUF):
            pltpu.make_async_copy(ones_v, agg_sh.at[dst_v.at[q0 + b]],
                                  sems[b]).wait()

            @pl.when(g < ngroups - 1)
            def _(b=b):
                pltpu.async_copy(ones_v, agg_sh.at[dst_v.at[q0 + b + NBUF]],
                                 sems[b], add=True)

        return carry

    lax.fori_loop(0, ngroups, group, 0)
    plsc.subcore_barrier()
    _readout(agg_sh, out_hbm, bounce_v, c, s, sems[:2], D2)


def _tc1_body(deg0_ref, deg1_ref, x_ref, w_ref, ya_ref, yb_ref, dinv_ref):
    d = deg0_ref[:, 0:1] + deg1_ref[:, 0:1]
    dinv = jnp.where(d > 0, lax.rsqrt(d), 0.0)
    xw = jnp.dot(x_ref[...], w_ref[...], preferred_element_type=jnp.float32)
    y = dinv * xw
    ya_ref[...] = y[:, :DH]
    yb_ref[...] = y[:, DH:]
    dinv_ref[...] = jnp.broadcast_to(dinv, (BLK, D2))


def _tc2_body(aa_ref, ab_ref, ya_ref, yb_ref, dinv_ref, w2_ref, b1_ref, y2_ref):
    a = jnp.concatenate([aa_ref[...] + ya_ref[...],
                         ab_ref[...] + yb_ref[...]], axis=1)
    dv = dinv_ref[:, 0:1]
    h = jnp.maximum(dv * a + b1_ref[...], 0.0)
    y2_ref[...] = dv * jnp.dot(h, w2_ref[...], preferred_element_type=jnp.float32)


def _tc3_body(a0_ref, a1_ref, y2_ref, dinv_ref, b2_ref, out_ref):
    z = dinv_ref[:, 0:1] * (a0_ref[...] + a1_ref[...] + y2_ref[...]) + b2_ref[...]
    z0 = z[:, 0:1]
    z1 = z[:, 1:2]
    m = jnp.maximum(z0, z1)
    e0 = jnp.exp(z0 - m)
    e1 = jnp.exp(z1 - m)
    den = e0 + e1
    out_ref[...] = jnp.concatenate(
        [e0 / den, e1 / den, jnp.zeros((BLK, D2 - 2), jnp.float32)], axis=1)


def _row_spec(d):
    return pl.BlockSpec((BLK, d), lambda i: (i, 0))


def _full_spec(shape):
    nd = len(shape)
    return pl.BlockSpec(shape, lambda i: (0,) * nd)


def kernel(x, edge_index, W1, b1, W2, b2):
    e32 = edge_index.astype(jnp.int32)
    src = e32[0]
    dst = e32[1]
    pad = TOTAL_E - src.shape[0]
    srcp = jnp.concatenate([src, jnp.zeros((pad,), jnp.int32)]).reshape(NCHUNK, CH)
    dstp = jnp.concatenate([dst, jnp.full((pad,), N, jnp.int32)]).reshape(NCHUNK, CH)
    xp = jnp.pad(x, ((0, NROWS - N), (0, 0)))
    w2p = jnp.pad(W2, ((0, 0), (0, D2 - W2.shape[1])))
    b1r = b1.reshape(1, D1)
    b2r = jnp.pad(b2, (0, D2 - 2)).reshape(1, D2)
    zeros_dh = jnp.zeros((NROWS, DH), jnp.float32)
    zeros_d2 = jnp.zeros((NROWS, D2), jnp.float32)
    ones_d2 = jnp.ones((CH, D2), jnp.float32)

    degp = _sc_deg(dstp, ones_d2, zeros_d2)
    deg0, deg1 = degp[:NROWS], degp[NROWS:]

    y1a, y1b, dinv = pl.pallas_call(
        _tc1_body,
        grid=(GRID,),
        in_specs=[_row_spec(D2), _row_spec(D2), _row_spec(D1),
                  _full_spec((D1, D1))],
        out_specs=[_row_spec(DH), _row_spec(DH), _row_spec(D2)],
        out_shape=[jax.ShapeDtypeStruct((NROWS, DH), jnp.float32),
                   jax.ShapeDtypeStruct((NROWS, DH), jnp.float32),
                   jax.ShapeDtypeStruct((NROWS, D2), jnp.float32)],
    )(deg0, deg1, xp, W1)

    agg1 = _sc_agg_l1(y1a, y1b, srcp, dstp, zeros_dh)
    agg1a, agg1b = agg1[:NROWS], agg1[NROWS:]

    y2 = pl.pallas_call(
        _tc2_body,
        grid=(GRID,),
        in_specs=[_row_spec(DH), _row_spec(DH), _row_spec(DH), _row_spec(DH),
                  _row_spec(D2), _full_spec((D1, D2)), _full_spec((1, D1))],
        out_specs=_row_spec(D2),
        out_shape=jax.ShapeDtypeStruct((NROWS, D2), jnp.float32),
    )(agg1a, agg1b, y1a, y1b, dinv, w2p, b1r)

    agg2 = _sc_agg_l2(y2, srcp, dstp, zeros_d2)
    a2_0, a2_1 = agg2[:NROWS], agg2[NROWS:]

    out = pl.pallas_call(
        _tc3_body,
        grid=(GRID,),
        in_specs=[_row_spec(D2), _row_spec(D2), _row_spec(D2), _row_spec(D2),
                  _full_spec((1, D2))],
        out_specs=_row_spec(D2),
        out_shape=jax.ShapeDtypeStruct((NROWS, D2), jnp.float32),
    )(a2_0, a2_1, y2, dinv, b2r)

    return out[:N, :2]
